# baseline (speedup 1.0000x reference)
import jax
import jax.numpy as jnp
from jax import lax
from jax.experimental import pallas as pl
from jax.experimental.pallas import tpu as pltpu
from jax.experimental.pallas import tpu_sc as plsc

_N = 10000
_E = 320000
_D = 128
_NC = 2
_NS = 16
_NW = _NC * _NS
_NP = 10240
_ZR = 64
_K = 80
_EPW = _E // _NW
_STEPS = _EPW // _K
_RPT = _NP // _NS


def _make_sc_agg(with_deg: bool, d: int = _D):
    mesh = plsc.VectorSubcoreMesh(core_axis_name="c", subcore_axis_name="s",
                                  num_cores=_NC, num_subcores=_NS)
    acc_type = jax.ShapeDtypeStruct((_NC * _NP, d), jnp.float32)
    out_type = [acc_type] if with_deg else acc_type
    scratch = [
        pltpu.VMEM((_K,), jnp.int32),
        pltpu.VMEM((_K,), jnp.int32),
        pltpu.VMEM((_K,), jnp.int32),
        pltpu.VMEM((_K,), jnp.int32),
        pltpu.VMEM((_K, d), jnp.float32),
        pltpu.VMEM((_K, d), jnp.float32),
        pltpu.VMEM((_ZR, d), jnp.float32),
        pltpu.VMEM_SHARED((_NP, d), jnp.float32),
        pltpu.SemaphoreType.DMA,
        pltpu.SemaphoreType.DMA,
        pltpu.SemaphoreType.DMA,
        pltpu.SemaphoreType.DMA,
        pltpu.SemaphoreType.DMA,
        pltpu.SemaphoreType.DMA,
        pltpu.SemaphoreType.DMA,
        pltpu.SemaphoreType.DMA,
    ]
    if with_deg:
        out_type.append(jax.ShapeDtypeStruct((_NC * _NP,), jnp.float32))
        scratch += [
            pltpu.VMEM((_K,), jnp.float32),
            pltpu.VMEM((_RPT,), jnp.float32),
            pltpu.VMEM_SHARED((_NP,), jnp.float32),
        ]

    def body(src, dst, h, *rest):
        if with_deg:
            (acc_out, deg_out, sidx_a, didx_a, sidx_b, didx_b, rows_a,
             rows_b, zblk, acc_sh, semg_a, semg_b, semis_a, semis_b,
             semid_a, semid_b, semsc_a, semsc_b, ones, zvec, deg_sh) = rest
        else:
            (acc_out, sidx_a, didx_a, sidx_b, didx_b, rows_a, rows_b,
             zblk, acc_sh, semg_a, semg_b, semis_a, semis_b,
             semid_a, semid_b, semsc_a, semsc_b) = rest

        c = lax.axis_index("c")
        s = lax.axis_index("s")
        wid = c * _NS + s

        z16 = jnp.zeros((16,), jnp.float32)

        def zrow(i, _):
            def zcol(j, carry):
                zblk[i, pl.ds(j * 16, 16)] = z16
                return carry
            return lax.fori_loop(0, d // 16, zcol, _)
        lax.fori_loop(0, _ZR, zrow, 0)

        if with_deg:
            o16 = jnp.ones((16,), jnp.float32)

            def fill_ones(j, carry):
                ones[pl.ds(j * 16, 16)] = o16
                return carry
            lax.fori_loop(0, _K // 16, fill_ones, 0)

            def fill_z(j, carry):
                zvec[pl.ds(j * 16, 16)] = z16
                return carry
            lax.fori_loop(0, _RPT // 16, fill_z, 0)

        r0 = s * _RPT
        zcps = [pltpu.async_copy(zblk, acc_sh.at[pl.ds(r0 + blk * _ZR, _ZR)],
                                 semg_a)
                for blk in range(_RPT // _ZR)]
        if with_deg:
            zcps.append(pltpu.async_copy(zvec, deg_sh.at[pl.ds(r0, _RPT)],
                                         semg_a))
        for cp in zcps:
            cp.wait()
        plsc.subcore_barrier()

        e0 = wid * _EPW

        def idx_src(c, buf, sem):
            return pltpu.make_async_copy(src.at[pl.ds(e0 + c * _K, _K)],
                                         buf, sem)

        def idx_dst(c, buf, sem):
            return pltpu.make_async_copy(dst.at[pl.ds(e0 + c * _K, _K)],
                                         buf, sem)

        def gath(sbuf, rbuf, sem):
            return pltpu.make_async_copy(h.at[sbuf], rbuf, sem)

        def scat(rbuf, dbuf, sem):
            return pltpu.async_copy(rbuf, acc_sh.at[dbuf], sem, add=True)

        def scat_deg(dbuf, sem):
            return pltpu.async_copy(ones, deg_sh.at[dbuf], sem, add=True)

        idx_src(0, sidx_a, semis_a).start()
        idx_dst(0, didx_a, semid_a).start()
        idx_src(1, sidx_b, semis_b).start()
        idx_dst(1, didx_b, semid_b).start()
        idx_src(0, sidx_a, semis_a).wait()
        gath(sidx_a, rows_a, semg_a).start()

        last = _STEPS - 1

        def pair(j, carry):
            ca = 2 * j
            cb = ca + 1
            cn_a = jnp.minimum(ca + 2, last)
            cn_b = jnp.minimum(cb + 2, last)
            idx_src(cb, sidx_b, semis_b).wait()
            gath(sidx_b, rows_b, semg_b).start()
            gath(sidx_a, rows_a, semg_a).wait()
            idx_src(cn_a, sidx_a, semis_a).start()
            idx_dst(ca, didx_a, semid_a).wait()
            sc_a = scat(rows_a, didx_a, semsc_a)
            dg_a = scat_deg(didx_a, semsc_a) if with_deg else None
            gath(sidx_b, rows_b, semg_b).wait()
            idx_src(cn_b, sidx_b, semis_b).start()
            idx_dst(cb, didx_b, semid_b).wait()
            sc_b = scat(rows_b, didx_b, semsc_b)
            dg_b = scat_deg(didx_b, semsc_b) if with_deg else None
            sc_a.wait()
            if with_deg:
                dg_a.wait()
            idx_dst(cn_a, didx_a, semid_a).start()
            idx_src(cn_a, sidx_a, semis_a).wait()
            gath(sidx_a, rows_a, semg_a).start()
            sc_b.wait()
            if with_deg:
                dg_b.wait()
            idx_dst(cn_b, didx_b, semid_b).start()
            return carry
        lax.fori_loop(0, _STEPS // 2, pair, 0)

        gath(sidx_a, rows_a, semg_a).wait()
        idx_dst(last, didx_a, semid_a).wait()
        pltpu.sync_copy(rows_a, acc_sh.at[didx_a], add=True)
        if with_deg:
            pltpu.sync_copy(ones, deg_sh.at[didx_a], add=True)
        idx_src(last, sidx_b, semis_b).wait()
        idx_dst(last, didx_b, semid_b).wait()
        plsc.subcore_barrier()

        o0 = c * _NP + r0
        ocps = [pltpu.async_copy(acc_sh.at[pl.ds(r0 + blk * _ZR, _ZR)],
                                 acc_out.at[pl.ds(o0 + blk * _ZR, _ZR), :],
                                 semg_b)
                for blk in range(_RPT // _ZR)]
        if with_deg:
            ocps.append(pltpu.async_copy(deg_sh.at[pl.ds(r0, _RPT)],
                                         deg_out.at[pl.ds(o0, _RPT)],
                                         semg_b))
        for cp in ocps:
            cp.wait()

    params = (pltpu.CompilerParams(use_tc_tiling_on_sc=False)
              if d % 128 else None)
    return pl.kernel(body, mesh=mesh, out_type=out_type,
                     scratch_types=scratch, compiler_params=params)


def _make_tc_layer(d_in: int, d_out: int, first: bool, ln: bool, act: bool,
                   proj: int = 0, pre_projected: bool = False,
                   bm: int = 512, n_rows: int = _NP):
    grid = (n_rows // bm,)
    d_agg = d_out if pre_projected else d_in

    def body(*refs):
        refs = list(refs)
        h_ref = refs.pop(0)
        a_ref = refs.pop(0)
        nd_ref = refs.pop(0)
        wl_ref = refs.pop(0)
        wr_ref = None if pre_projected else refs.pop(0)
        b_ref = refs.pop(0)
        wp_ref = refs.pop(0) if proj else None
        o_ref = refs.pop(0)
        n_ref = refs.pop(0) if first else None
        p_ref = refs.pop(0) if proj else None
        if first:
            deg = nd_ref[0] + nd_ref[1]
            nrm = jnp.where(deg > 0, 1.0 / deg, 0.0)
            n_ref[...] = nrm
        else:
            nrm = nd_ref[...]
        ah = (a_ref[0] + a_ref[1]) * nrm
        out = jnp.dot(h_ref[...], wl_ref[...],
                      preferred_element_type=jnp.float32) + b_ref[...]
        if pre_projected:
            out = out + ah
        else:
            out = out + jnp.dot(ah, wr_ref[...],
                                preferred_element_type=jnp.float32)
        if ln:
            mu = jnp.mean(out, axis=1, keepdims=True)
            ctr = out - mu
            var = jnp.mean(ctr * ctr, axis=1, keepdims=True)
            out = ctr * lax.rsqrt(var + 1e-5)
        if act:
            out = jnp.maximum(out, 0.0)
        o_ref[...] = out
        if proj:
            p_ref[...] = jnp.dot(out, wp_ref[...],
                                 preferred_element_type=jnp.float32)

    in_specs = [
        pl.BlockSpec((bm, d_in), lambda i: (i, 0)),
        pl.BlockSpec((2, bm, d_agg), lambda i: (0, i, 0)),
        (pl.BlockSpec((2, bm, 1), lambda i: (0, i, 0)) if first
         else pl.BlockSpec((bm, 1), lambda i: (i, 0))),
        pl.BlockSpec((d_in, d_out), lambda i: (0, 0)),
    ]
    if not pre_projected:
        in_specs.append(pl.BlockSpec((d_in, d_out), lambda i: (0, 0)))
    in_specs.append(pl.BlockSpec((1, d_out), lambda i: (0, 0)))
    if proj:
        in_specs.append(pl.BlockSpec((d_out, proj), lambda i: (0, 0)))
    out_specs = [pl.BlockSpec((bm, d_out), lambda i: (i, 0))]
    out_shape = [jax.ShapeDtypeStruct((n_rows, d_out), jnp.float32)]
    if first:
        out_specs.append(pl.BlockSpec((bm, 1), lambda i: (i, 0)))
        out_shape.append(jax.ShapeDtypeStruct((n_rows, 1), jnp.float32))
    if proj:
        out_specs.append(pl.BlockSpec((bm, proj), lambda i: (i, 0)))
        out_shape.append(jax.ShapeDtypeStruct((n_rows, proj), jnp.float32))
    return pl.pallas_call(body, grid=grid, in_specs=in_specs,
                          out_specs=out_specs, out_shape=out_shape)


import functools as _functools

_make_sc_agg = _functools.cache(_make_sc_agg)
_LAYER1 = _make_tc_layer(_D, _D, first=True, ln=True, act=True,
                         bm=400, n_rows=_N)
_LAYER2 = _make_tc_layer(_D, _D, first=False, ln=True, act=True, proj=64,
                         bm=400, n_rows=_N)
_LAYER3 = _make_tc_layer(_D, 64, first=False, ln=False, act=False,
                         pre_projected=True, bm=400, n_rows=_N)


def kernel(x, edge_index, W1, b1, W2, b2, W3, b3):
    src = edge_index[0]
    dst = edge_index[1]
    accf, degf = _make_sc_agg(True)(src, dst, x)
    acc1 = accf.reshape(_NC, _NP, _D)
    deg3 = degf.reshape(_NC, _NP, 1)
    h1, norm = _LAYER1(x, acc1, deg3, W1[:, :_D].T, W1[:, _D:].T,
                       b1.reshape(1, -1))
    acc2 = _make_sc_agg(False)(src, dst, h1).reshape(_NC, _NP, _D)
    h2, p2 = _LAYER2(h1, acc2, norm, W2[:, :_D].T, W2[:, _D:].T,
                     b2.reshape(1, -1), W3[:, _D:].T)
    acc3 = _make_sc_agg(False, 64)(src, dst, p2).reshape(_NC, _NP, 64)
    (h3,) = _LAYER3(h2, acc3, norm, W3[:, :_D].T, b3.reshape(1, -1))
    return h3


# speedup vs baseline: 1.0395x; 1.0395x over previous
import jax
import jax.numpy as jnp
from jax import lax
from jax.experimental import pallas as pl
from jax.experimental.pallas import tpu as pltpu
from jax.experimental.pallas import tpu_sc as plsc

_N = 10000
_E = 320000
_D = 128
_NC = 2
_NS = 16
_NW = _NC * _NS
_NP = 10240
_ZR = 64
_K = 128
_EPW = _E // _NW
_EPWP = 10240
_PAD = _EPWP - _EPW
_STEPS = _EPWP // _K
_RPT = _NP // _NS


def _make_sc_agg(with_deg: bool, d: int = _D):
    mesh = plsc.VectorSubcoreMesh(core_axis_name="c", subcore_axis_name="s",
                                  num_cores=_NC, num_subcores=_NS)
    acc_type = jax.ShapeDtypeStruct((_NC * _NP, d), jnp.float32)
    out_type = [acc_type] if with_deg else acc_type
    scratch = [
        pltpu.VMEM((_K,), jnp.int32),
        pltpu.VMEM((_K,), jnp.int32),
        pltpu.VMEM((_K,), jnp.int32),
        pltpu.VMEM((_K,), jnp.int32),
        pltpu.VMEM((_K, d), jnp.float32),
        pltpu.VMEM((_K, d), jnp.float32),
        pltpu.VMEM((_ZR, d), jnp.float32),
        pltpu.VMEM_SHARED((_NP, d), jnp.float32),
        pltpu.SemaphoreType.DMA,
        pltpu.SemaphoreType.DMA,
        pltpu.SemaphoreType.DMA,
        pltpu.SemaphoreType.DMA,
        pltpu.SemaphoreType.DMA,
        pltpu.SemaphoreType.DMA,
        pltpu.SemaphoreType.DMA,
        pltpu.SemaphoreType.DMA,
    ]
    if with_deg:
        out_type.append(jax.ShapeDtypeStruct((_NC * _NP,), jnp.float32))
        scratch += [
            pltpu.VMEM((_K,), jnp.float32),
            pltpu.VMEM((_RPT,), jnp.float32),
            pltpu.VMEM_SHARED((_NP,), jnp.float32),
        ]

    def body(src, dst, h, *rest):
        if with_deg:
            (acc_out, deg_out, sidx_a, didx_a, sidx_b, didx_b, rows_a,
             rows_b, zblk, acc_sh, semg_a, semg_b, semis_a, semis_b,
             semid_a, semid_b, semsc_a, semsc_b, ones, zvec, deg_sh) = rest
        else:
            (acc_out, sidx_a, didx_a, sidx_b, didx_b, rows_a, rows_b,
             zblk, acc_sh, semg_a, semg_b, semis_a, semis_b,
             semid_a, semid_b, semsc_a, semsc_b) = rest

        c = lax.axis_index("c")
        s = lax.axis_index("s")
        wid = c * _NS + s

        z16 = jnp.zeros((16,), jnp.float32)

        def zrow(i, _):
            def zcol(j, carry):
                zblk[i, pl.ds(j * 16, 16)] = z16
                return carry
            return lax.fori_loop(0, d // 16, zcol, _)
        lax.fori_loop(0, _ZR, zrow, 0)

        if with_deg:
            o16 = jnp.ones((16,), jnp.float32)

            def fill_ones(j, carry):
                ones[pl.ds(j * 16, 16)] = o16
                return carry
            lax.fori_loop(0, _K // 16, fill_ones, 0)

            def fill_z(j, carry):
                zvec[pl.ds(j * 16, 16)] = z16
                return carry
            lax.fori_loop(0, _RPT // 16, fill_z, 0)

        r0 = s * _RPT
        zcps = [pltpu.async_copy(zblk, acc_sh.at[pl.ds(r0 + blk * _ZR, _ZR)],
                                 semg_a)
                for blk in range(_RPT // _ZR)]
        if with_deg:
            zcps.append(pltpu.async_copy(zvec, deg_sh.at[pl.ds(r0, _RPT)],
                                         semg_a))
        for cp in zcps:
            cp.wait()
        plsc.subcore_barrier()

        e0 = wid * _EPWP

        def idx_src(c, buf, sem):
            return pltpu.make_async_copy(src.at[pl.ds(e0 + c * _K, _K)],
                                         buf, sem)

        def idx_dst(c, buf, sem):
            return pltpu.make_async_copy(dst.at[pl.ds(e0 + c * _K, _K)],
                                         buf, sem)

        def gath(sbuf, rbuf, sem):
            return pltpu.make_async_copy(h.at[sbuf], rbuf, sem)

        def scat(rbuf, dbuf, sem):
            return pltpu.async_copy(rbuf, acc_sh.at[dbuf], sem, add=True)

        def scat_deg(dbuf, sem):
            return pltpu.async_copy(ones, deg_sh.at[dbuf], sem, add=True)

        idx_src(0, sidx_a, semis_a).start()
        idx_dst(0, didx_a, semid_a).start()
        idx_src(1, sidx_b, semis_b).start()
        idx_dst(1, didx_b, semid_b).start()
        idx_src(0, sidx_a, semis_a).wait()
        gath(sidx_a, rows_a, semg_a).start()

        last = _STEPS - 1

        def pair(j, carry):
            ca = 2 * j
            cb = ca + 1
            cn_a = jnp.minimum(ca + 2, last)
            cn_b = jnp.minimum(cb + 2, last)
            idx_src(cb, sidx_b, semis_b).wait()
            gath(sidx_b, rows_b, semg_b).start()
            gath(sidx_a, rows_a, semg_a).wait()
            idx_src(cn_a, sidx_a, semis_a).start()
            idx_dst(ca, didx_a, semid_a).wait()
            sc_a = scat(rows_a, didx_a, semsc_a)
            dg_a = scat_deg(didx_a, semsc_a) if with_deg else None
            gath(sidx_b, rows_b, semg_b).wait()
            idx_src(cn_b, sidx_b, semis_b).start()
            idx_dst(cb, didx_b, semid_b).wait()
            sc_b = scat(rows_b, didx_b, semsc_b)
            dg_b = scat_deg(didx_b, semsc_b) if with_deg else None
            sc_a.wait()
            if with_deg:
                dg_a.wait()
            idx_dst(cn_a, didx_a, semid_a).start()
            idx_src(cn_a, sidx_a, semis_a).wait()
            gath(sidx_a, rows_a, semg_a).start()
            sc_b.wait()
            if with_deg:
                dg_b.wait()
            idx_dst(cn_b, didx_b, semid_b).start()
            return carry
        lax.fori_loop(0, _STEPS // 2, pair, 0)

        gath(sidx_a, rows_a, semg_a).wait()
        idx_dst(last, didx_a, semid_a).wait()
        idx_src(last, sidx_b, semis_b).wait()
        idx_dst(last, didx_b, semid_b).wait()
        plsc.subcore_barrier()

        o0 = c * _NP + r0
        ocps = [pltpu.async_copy(acc_sh.at[pl.ds(r0 + blk * _ZR, _ZR)],
                                 acc_out.at[pl.ds(o0 + blk * _ZR, _ZR), :],
                                 semg_b)
                for blk in range(_RPT // _ZR)]
        if with_deg:
            ocps.append(pltpu.async_copy(deg_sh.at[pl.ds(r0, _RPT)],
                                         deg_out.at[pl.ds(o0, _RPT)],
                                         semg_b))
        for cp in ocps:
            cp.wait()

    params = (pltpu.CompilerParams(use_tc_tiling_on_sc=False)
              if d % 128 else None)
    return pl.kernel(body, mesh=mesh, out_type=out_type,
                     scratch_types=scratch, compiler_params=params)


def _make_tc_layer(d_in: int, d_out: int, first: bool, ln: bool, act: bool,
                   proj: int = 0, pre_projected: bool = False,
                   bm: int = 512, n_rows: int = _NP):
    grid = (n_rows // bm,)
    d_agg = d_out if pre_projected else d_in

    def body(*refs):
        refs = list(refs)
        h_ref = refs.pop(0)
        a_ref = refs.pop(0)
        nd_ref = refs.pop(0)
        wl_ref = refs.pop(0)
        wr_ref = None if pre_projected else refs.pop(0)
        b_ref = refs.pop(0)
        wp_ref = refs.pop(0) if proj else None
        o_ref = refs.pop(0)
        n_ref = refs.pop(0) if first else None
        p_ref = refs.pop(0) if proj else None
        if first:
            deg = nd_ref[0] + nd_ref[1]
            nrm = jnp.where(deg > 0, 1.0 / deg, 0.0)
            n_ref[...] = nrm
        else:
            nrm = nd_ref[...]
        ah = (a_ref[0] + a_ref[1]) * nrm
        out = jnp.dot(h_ref[...], wl_ref[...],
                      preferred_element_type=jnp.float32) + b_ref[...]
        if pre_projected:
            out = out + ah
        else:
            out = out + jnp.dot(ah, wr_ref[...],
                                preferred_element_type=jnp.float32)
        if ln:
            mu = jnp.mean(out, axis=1, keepdims=True)
            ctr = out - mu
            var = jnp.mean(ctr * ctr, axis=1, keepdims=True)
            out = ctr * lax.rsqrt(var + 1e-5)
        if act:
            out = jnp.maximum(out, 0.0)
        o_ref[...] = out
        if proj:
            p_ref[...] = jnp.dot(out, wp_ref[...],
                                 preferred_element_type=jnp.float32)

    in_specs = [
        pl.BlockSpec((bm, d_in), lambda i: (i, 0)),
        pl.BlockSpec((2, bm, d_agg), lambda i: (0, i, 0)),
        (pl.BlockSpec((2, bm, 1), lambda i: (0, i, 0)) if first
         else pl.BlockSpec((bm, 1), lambda i: (i, 0))),
        pl.BlockSpec((d_in, d_out), lambda i: (0, 0)),
    ]
    if not pre_projected:
        in_specs.append(pl.BlockSpec((d_in, d_out), lambda i: (0, 0)))
    in_specs.append(pl.BlockSpec((1, d_out), lambda i: (0, 0)))
    if proj:
        in_specs.append(pl.BlockSpec((d_out, proj), lambda i: (0, 0)))
    out_specs = [pl.BlockSpec((bm, d_out), lambda i: (i, 0))]
    out_shape = [jax.ShapeDtypeStruct((n_rows, d_out), jnp.float32)]
    if first:
        out_specs.append(pl.BlockSpec((bm, 1), lambda i: (i, 0)))
        out_shape.append(jax.ShapeDtypeStruct((n_rows, 1), jnp.float32))
    if proj:
        out_specs.append(pl.BlockSpec((bm, proj), lambda i: (i, 0)))
        out_shape.append(jax.ShapeDtypeStruct((n_rows, proj), jnp.float32))
    return pl.pallas_call(body, grid=grid, in_specs=in_specs,
                          out_specs=out_specs, out_shape=out_shape)


import functools as _functools

_make_sc_agg = _functools.cache(_make_sc_agg)
_LAYER1 = _make_tc_layer(_D, _D, first=True, ln=True, act=True,
                         bm=400, n_rows=_N)
_LAYER2 = _make_tc_layer(_D, _D, first=False, ln=True, act=True, proj=64,
                         bm=400, n_rows=_N)
_LAYER3 = _make_tc_layer(_D, 64, first=False, ln=False, act=False,
                         pre_projected=True, bm=400, n_rows=_N)


def kernel(x, edge_index, W1, b1, W2, b2, W3, b3):
    pad_ids = jnp.arange(_PAD, dtype=jnp.int32)
    fake_src = jnp.broadcast_to(pad_ids, (_NW, _PAD))
    fake_dst = jnp.broadcast_to(_N + pad_ids, (_NW, _PAD))
    src = jnp.concatenate(
        [edge_index[0].reshape(_NW, _EPW), fake_src], axis=1).reshape(-1)
    dst = jnp.concatenate(
        [edge_index[1].reshape(_NW, _EPW), fake_dst], axis=1).reshape(-1)
    accf, degf = _make_sc_agg(True)(src, dst, x)
    acc1 = accf.reshape(_NC, _NP, _D)
    deg3 = degf.reshape(_NC, _NP, 1)
    h1, norm = _LAYER1(x, acc1, deg3, W1[:, :_D].T, W1[:, _D:].T,
                       b1.reshape(1, -1))
    acc2 = _make_sc_agg(False)(src, dst, h1).reshape(_NC, _NP, _D)
    h2, p2 = _LAYER2(h1, acc2, norm, W2[:, :_D].T, W2[:, _D:].T,
                     b2.reshape(1, -1), W3[:, _D:].T)
    acc3 = _make_sc_agg(False, 64)(src, dst, p2).reshape(_NC, _NP, 64)
    (h3,) = _LAYER3(h2, acc3, norm, W3[:, :_D].T, b3.reshape(1, -1))
    return h3


# speedup vs baseline: 1.0595x; 1.0192x over previous
import jax
import jax.numpy as jnp
from jax import lax
from jax.experimental import pallas as pl
from jax.experimental.pallas import tpu as pltpu
from jax.experimental.pallas import tpu_sc as plsc

_N = 10000
_E = 320000
_D = 128
_NC = 2
_NS = 16
_NW = _NC * _NS
_NP = 10240
_ZR = 64
_K = 128
_EPW = _E // _NW
_STEPS = _EPW // _K
_KT = _EPW - _STEPS * _K
_RPT = _NP // _NS


def _make_sc_agg(with_deg: bool, d: int = _D):
    mesh = plsc.VectorSubcoreMesh(core_axis_name="c", subcore_axis_name="s",
                                  num_cores=_NC, num_subcores=_NS)
    acc_type = jax.ShapeDtypeStruct((_NC, _NP, d), jnp.float32)
    out_type = [acc_type] if with_deg else acc_type
    scratch = [
        pltpu.VMEM((_K,), jnp.int32),
        pltpu.VMEM((_K,), jnp.int32),
        pltpu.VMEM((_K,), jnp.int32),
        pltpu.VMEM((_K,), jnp.int32),
        pltpu.VMEM((_K, d), jnp.float32),
        pltpu.VMEM((_K, d), jnp.float32),
        pltpu.VMEM((_KT,), jnp.int32),
        pltpu.VMEM((_KT,), jnp.int32),
        pltpu.VMEM((_KT, d), jnp.float32),
        pltpu.VMEM((_ZR, d), jnp.float32),
        pltpu.VMEM_SHARED((_NP, d), jnp.float32),
        pltpu.SemaphoreType.DMA,
        pltpu.SemaphoreType.DMA,
        pltpu.SemaphoreType.DMA,
        pltpu.SemaphoreType.DMA,
        pltpu.SemaphoreType.DMA,
        pltpu.SemaphoreType.DMA,
        pltpu.SemaphoreType.DMA,
        pltpu.SemaphoreType.DMA,
    ]
    if with_deg:
        out_type.append(jax.ShapeDtypeStruct((_NC * _NP,), jnp.float32))
        scratch += [
            pltpu.VMEM((_K,), jnp.float32),
            pltpu.VMEM((_RPT,), jnp.float32),
            pltpu.VMEM_SHARED((_NP,), jnp.float32),
        ]

    def body(src, dst, h, *rest):
        if with_deg:
            (acc_out, deg_out, sidx_a, didx_a, sidx_b, didx_b, rows_a,
             rows_b, sidx_t, didx_t, rows_t, zblk, acc_sh, semg_a, semg_b,
             semis_a, semis_b, semid_a, semid_b, semsc_a, semsc_b,
             ones, zvec, deg_sh) = rest
        else:
            (acc_out, sidx_a, didx_a, sidx_b, didx_b, rows_a, rows_b,
             sidx_t, didx_t, rows_t, zblk, acc_sh, semg_a, semg_b,
             semis_a, semis_b, semid_a, semid_b, semsc_a, semsc_b) = rest

        c = lax.axis_index("c")
        s = lax.axis_index("s")
        wid = c * _NS + s

        z16 = jnp.zeros((16,), jnp.float32)

        def zrow(i, _):
            def zcol(j, carry):
                zblk[i, pl.ds(j * 16, 16)] = z16
                return carry
            return lax.fori_loop(0, d // 16, zcol, _)
        lax.fori_loop(0, _ZR, zrow, 0)

        if with_deg:
            o16 = jnp.ones((16,), jnp.float32)

            def fill_ones(j, carry):
                ones[pl.ds(j * 16, 16)] = o16
                return carry
            lax.fori_loop(0, _K // 16, fill_ones, 0)

            def fill_z(j, carry):
                zvec[pl.ds(j * 16, 16)] = z16
                return carry
            lax.fori_loop(0, _RPT // 16, fill_z, 0)

        r0 = s * _RPT
        zcps = [pltpu.async_copy(zblk, acc_sh.at[pl.ds(r0 + blk * _ZR, _ZR)],
                                 semg_a)
                for blk in range(_RPT // _ZR)]
        if with_deg:
            zcps.append(pltpu.async_copy(zvec, deg_sh.at[pl.ds(r0, _RPT)],
                                         semg_a))
        for cp in zcps:
            cp.wait()
        plsc.subcore_barrier()

        e0 = wid * _EPW

        def idx_src(c, buf, sem):
            return pltpu.make_async_copy(src.at[pl.ds(e0 + c * _K, _K)],
                                         buf, sem)

        def idx_dst(c, buf, sem):
            return pltpu.make_async_copy(dst.at[pl.ds(e0 + c * _K, _K)],
                                         buf, sem)

        def gath(sbuf, rbuf, sem):
            return pltpu.make_async_copy(h.at[sbuf], rbuf, sem)

        def scat(rbuf, dbuf, sem):
            return pltpu.async_copy(rbuf, acc_sh.at[dbuf], sem, add=True)

        def scat_deg(dbuf, sem):
            return pltpu.async_copy(ones, deg_sh.at[dbuf], sem, add=True)

        idx_src(0, sidx_a, semis_a).start()
        idx_dst(0, didx_a, semid_a).start()
        idx_src(1, sidx_b, semis_b).start()
        idx_dst(1, didx_b, semid_b).start()
        idx_src(0, sidx_a, semis_a).wait()
        gath(sidx_a, rows_a, semg_a).start()

        last = _STEPS - 1

        def pair(j, carry):
            ca = 2 * j
            cb = ca + 1
            cn_a = jnp.minimum(ca + 2, last)
            cn_b = jnp.minimum(cb + 2, last)
            idx_src(cb, sidx_b, semis_b).wait()
            gath(sidx_b, rows_b, semg_b).start()
            gath(sidx_a, rows_a, semg_a).wait()
            idx_src(cn_a, sidx_a, semis_a).start()
            idx_dst(ca, didx_a, semid_a).wait()
            sc_a = scat(rows_a, didx_a, semsc_a)
            dg_a = scat_deg(didx_a, semsc_a) if with_deg else None
            gath(sidx_b, rows_b, semg_b).wait()
            idx_src(cn_b, sidx_b, semis_b).start()
            idx_dst(cb, didx_b, semid_b).wait()
            sc_b = scat(rows_b, didx_b, semsc_b)
            dg_b = scat_deg(didx_b, semsc_b) if with_deg else None
            sc_a.wait()
            if with_deg:
                dg_a.wait()
            idx_dst(cn_a, didx_a, semid_a).start()
            idx_src(cn_a, sidx_a, semis_a).wait()
            gath(sidx_a, rows_a, semg_a).start()
            sc_b.wait()
            if with_deg:
                dg_b.wait()
            idx_dst(cn_b, didx_b, semid_b).start()
            return carry
        lax.fori_loop(0, _STEPS // 2, pair, 0)

        pltpu.sync_copy(src.at[pl.ds(e0 + _STEPS * _K, _KT)], sidx_t)
        pltpu.sync_copy(dst.at[pl.ds(e0 + _STEPS * _K, _KT)], didx_t)
        pltpu.async_copy(h.at[sidx_t], rows_t, semg_b).wait()
        pltpu.sync_copy(rows_t, acc_sh.at[didx_t], add=True)
        if with_deg:
            pltpu.sync_copy(ones.at[pl.ds(0, _KT)], deg_sh.at[didx_t],
                            add=True)
        gath(sidx_a, rows_a, semg_a).wait()
        idx_dst(last, didx_a, semid_a).wait()
        idx_src(last, sidx_b, semis_b).wait()
        idx_dst(last, didx_b, semid_b).wait()
        plsc.subcore_barrier()

        ocps = [pltpu.async_copy(acc_sh.at[pl.ds(r0 + blk * _ZR, _ZR)],
                                 acc_out.at[c, pl.ds(r0 + blk * _ZR, _ZR), :],
                                 semg_b)
                for blk in range(_RPT // _ZR)]
        if with_deg:
            ocps.append(pltpu.async_copy(deg_sh.at[pl.ds(r0, _RPT)],
                                         deg_out.at[pl.ds(c * _NP + r0, _RPT)],
                                         semg_b))
        for cp in ocps:
            cp.wait()

    params = (pltpu.CompilerParams(use_tc_tiling_on_sc=False)
              if d % 128 else None)
    return pl.kernel(body, mesh=mesh, out_type=out_type,
                     scratch_types=scratch, compiler_params=params)


def _make_tc_layer(d_in: int, d_out: int, first: bool, ln: bool, act: bool,
                   proj: int = 0, pre_projected: bool = False,
                   bm: int = 512, n_rows: int = _NP):
    grid = (n_rows // bm,)
    d_agg = d_out if pre_projected else d_in

    def body(*refs):
        refs = list(refs)
        h_ref = refs.pop(0)
        a_ref = refs.pop(0)
        nd_ref = refs.pop(0)
        wl_ref = refs.pop(0)
        wr_ref = None if pre_projected else refs.pop(0)
        b_ref = refs.pop(0)
        wp_ref = refs.pop(0) if proj else None
        o_ref = refs.pop(0)
        n_ref = refs.pop(0) if first else None
        p_ref = refs.pop(0) if proj else None
        if first:
            deg = nd_ref[0] + nd_ref[1]
            nrm = jnp.where(deg > 0, 1.0 / deg, 0.0)
            n_ref[...] = nrm
        else:
            nrm = nd_ref[...]
        ah = (a_ref[0] + a_ref[1]) * nrm
        out = jnp.dot(h_ref[...], wl_ref[...],
                      preferred_element_type=jnp.float32) + b_ref[...]
        if pre_projected:
            out = out + ah
        else:
            out = out + jnp.dot(ah, wr_ref[...],
                                preferred_element_type=jnp.float32)
        if ln:
            mu = jnp.mean(out, axis=1, keepdims=True)
            ctr = out - mu
            var = jnp.mean(ctr * ctr, axis=1, keepdims=True)
            out = ctr * lax.rsqrt(var + 1e-5)
        if act:
            out = jnp.maximum(out, 0.0)
        o_ref[...] = out
        if proj:
            p_ref[...] = jnp.dot(out, wp_ref[...],
                                 preferred_element_type=jnp.float32)

    in_specs = [
        pl.BlockSpec((bm, d_in), lambda i: (i, 0)),
        pl.BlockSpec((2, bm, d_agg), lambda i: (0, i, 0)),
        (pl.BlockSpec((2, bm, 1), lambda i: (0, i, 0)) if first
         else pl.BlockSpec((bm, 1), lambda i: (i, 0))),
        pl.BlockSpec((d_in, d_out), lambda i: (0, 0)),
    ]
    if not pre_projected:
        in_specs.append(pl.BlockSpec((d_in, d_out), lambda i: (0, 0)))
    in_specs.append(pl.BlockSpec((1, d_out), lambda i: (0, 0)))
    if proj:
        in_specs.append(pl.BlockSpec((d_out, proj), lambda i: (0, 0)))
    out_specs = [pl.BlockSpec((bm, d_out), lambda i: (i, 0))]
    out_shape = [jax.ShapeDtypeStruct((n_rows, d_out), jnp.float32)]
    if first:
        out_specs.append(pl.BlockSpec((bm, 1), lambda i: (i, 0)))
        out_shape.append(jax.ShapeDtypeStruct((n_rows, 1), jnp.float32))
    if proj:
        out_specs.append(pl.BlockSpec((bm, proj), lambda i: (i, 0)))
        out_shape.append(jax.ShapeDtypeStruct((n_rows, proj), jnp.float32))
    return pl.pallas_call(body, grid=grid, in_specs=in_specs,
                          out_specs=out_specs, out_shape=out_shape)


import functools as _functools

_make_sc_agg = _functools.cache(_make_sc_agg)
_LAYER1 = _make_tc_layer(_D, _D, first=True, ln=True, act=True,
                         bm=400, n_rows=_N)
_LAYER2 = _make_tc_layer(_D, _D, first=False, ln=True, act=True, proj=64,
                         bm=400, n_rows=_N)
_LAYER3 = _make_tc_layer(_D, 64, first=False, ln=False, act=False,
                         pre_projected=True, bm=400, n_rows=_N)


def kernel(x, edge_index, W1, b1, W2, b2, W3, b3):
    src = edge_index[0]
    dst = edge_index[1]
    acc1, degf = _make_sc_agg(True)(src, dst, x)
    deg3 = degf.reshape(_NC, _NP, 1)
    h1, norm = _LAYER1(x, acc1, deg3, W1[:, :_D].T, W1[:, _D:].T,
                       b1.reshape(1, -1))
    acc2 = _make_sc_agg(False)(src, dst, h1)
    h2, p2 = _LAYER2(h1, acc2, norm, W2[:, :_D].T, W2[:, _D:].T,
                     b2.reshape(1, -1), W3[:, _D:].T)
    acc3 = _make_sc_agg(False, 64)(src, dst, p2)
    (h3,) = _LAYER3(h2, acc3, norm, W3[:, :_D].T, b3.reshape(1, -1))
    return h3


# speedup vs baseline: 1.0604x; 1.0009x over previous
import jax
import jax.numpy as jnp
from jax import lax
from jax.experimental import pallas as pl
from jax.experimental.pallas import tpu as pltpu
from jax.experimental.pallas import tpu_sc as plsc

_N = 10000
_E = 320000
_D = 128
_NC = 2
_NS = 16
_NW = _NC * _NS
_NP = 10240
_ZR = 64
_K = 128
_EPW = _E // _NW
_STEPS = _EPW // _K
_KT = _EPW - _STEPS * _K
_RPT = _NP // _NS


def _make_sc_agg(with_deg: bool, d: int = _D):
    mesh = plsc.VectorSubcoreMesh(core_axis_name="c", subcore_axis_name="s",
                                  num_cores=_NC, num_subcores=_NS)
    acc_type = jax.ShapeDtypeStruct((_NC, _NP, d), jnp.float32)
    out_type = [acc_type] if with_deg else acc_type
    scratch = [
        pltpu.VMEM((_K,), jnp.int32),
        pltpu.VMEM((_K,), jnp.int32),
        pltpu.VMEM((_K,), jnp.int32),
        pltpu.VMEM((_K,), jnp.int32),
        pltpu.VMEM((_K, d), jnp.float32),
        pltpu.VMEM((_K, d), jnp.float32),
        pltpu.VMEM((_KT,), jnp.int32),
        pltpu.VMEM((_KT,), jnp.int32),
        pltpu.VMEM((_KT, d), jnp.float32),
        pltpu.VMEM((_ZR, d), jnp.float32),
        pltpu.VMEM_SHARED((_NP, d), jnp.float32),
        pltpu.SemaphoreType.DMA,
        pltpu.SemaphoreType.DMA,
        pltpu.SemaphoreType.DMA,
        pltpu.SemaphoreType.DMA,
        pltpu.SemaphoreType.DMA,
        pltpu.SemaphoreType.DMA,
        pltpu.SemaphoreType.DMA,
        pltpu.SemaphoreType.DMA,
    ]
    if with_deg:
        out_type.append(jax.ShapeDtypeStruct((_NC * _NP,), jnp.float32))
        scratch += [
            pltpu.VMEM((_K,), jnp.float32),
            pltpu.VMEM((_RPT,), jnp.float32),
            pltpu.VMEM_SHARED((_NP,), jnp.float32),
        ]

    def body(src, dst, h, *rest):
        if with_deg:
            (acc_out, deg_out, sidx_a, didx_a, sidx_b, didx_b, rows_a,
             rows_b, sidx_t, didx_t, rows_t, zblk, acc_sh, semg_a, semg_b,
             semis_a, semis_b, semid_a, semid_b, semsc_a, semsc_b,
             ones, zvec, deg_sh) = rest
        else:
            (acc_out, sidx_a, didx_a, sidx_b, didx_b, rows_a, rows_b,
             sidx_t, didx_t, rows_t, zblk, acc_sh, semg_a, semg_b,
             semis_a, semis_b, semid_a, semid_b, semsc_a, semsc_b) = rest

        c = lax.axis_index("c")
        s = lax.axis_index("s")
        wid = c * _NS + s

        z16 = jnp.zeros((16,), jnp.float32)

        def zrow(i, _):
            def zcol(j, carry):
                zblk[i, pl.ds(j * 16, 16)] = z16
                return carry
            return lax.fori_loop(0, d // 16, zcol, _)
        lax.fori_loop(0, _ZR, zrow, 0)

        if with_deg:
            o16 = jnp.ones((16,), jnp.float32)

            def fill_ones(j, carry):
                ones[pl.ds(j * 16, 16)] = o16
                return carry
            lax.fori_loop(0, _K // 16, fill_ones, 0)

            def fill_z(j, carry):
                zvec[pl.ds(j * 16, 16)] = z16
                return carry
            lax.fori_loop(0, _RPT // 16, fill_z, 0)

        r0 = s * _RPT
        zcps = [pltpu.async_copy(zblk, acc_sh.at[pl.ds(r0 + blk * _ZR, _ZR)],
                                 semg_a)
                for blk in range(_RPT // _ZR)]
        if with_deg:
            zcps.append(pltpu.async_copy(zvec, deg_sh.at[pl.ds(r0, _RPT)],
                                         semg_a))
        for cp in zcps:
            cp.wait()
        plsc.subcore_barrier()

        e0 = wid * _EPW

        def idx_src(ch, buf, sem):
            return pltpu.make_async_copy(src.at[pl.ds(e0 + ch * _K, _K)],
                                         buf, sem)

        def idx_dst(ch, buf, sem):
            return pltpu.make_async_copy(dst.at[pl.ds(e0 + ch * _K, _K)],
                                         buf, sem)

        def gath(sbuf, rbuf, sem):
            return pltpu.make_async_copy(h.at[sbuf], rbuf, sem)

        def scat(rbuf, dbuf, sem):
            return pltpu.async_copy(rbuf, acc_sh.at[dbuf], sem, add=True)

        def scat_deg(dbuf, sem):
            return pltpu.async_copy(ones, deg_sh.at[dbuf], sem, add=True)

        idx_src(0, sidx_a, semis_a).start()
        idx_dst(0, didx_a, semid_a).start()
        idx_src(1, sidx_b, semis_b).start()
        idx_dst(1, didx_b, semid_b).start()
        idx_src(0, sidx_a, semis_a).wait()
        gath(sidx_a, rows_a, semg_a).start()

        last = _STEPS - 1

        def pair(j, carry):
            ca = 2 * j
            cb = ca + 1
            cn_a = jnp.minimum(ca + 2, last)
            cn_b = jnp.minimum(cb + 2, last)
            idx_src(cb, sidx_b, semis_b).wait()
            gath(sidx_b, rows_b, semg_b).start()
            gath(sidx_a, rows_a, semg_a).wait()
            idx_src(cn_a, sidx_a, semis_a).start()
            idx_dst(ca, didx_a, semid_a).wait()
            sc_a = scat(rows_a, didx_a, semsc_a)
            dg_a = scat_deg(didx_a, semsc_a) if with_deg else None
            gath(sidx_b, rows_b, semg_b).wait()
            idx_src(cn_b, sidx_b, semis_b).start()
            idx_dst(cb, didx_b, semid_b).wait()
            sc_b = scat(rows_b, didx_b, semsc_b)
            dg_b = scat_deg(didx_b, semsc_b) if with_deg else None
            sc_a.wait()
            if with_deg:
                dg_a.wait()
            idx_dst(cn_a, didx_a, semid_a).start()
            idx_src(cn_a, sidx_a, semis_a).wait()
            gath(sidx_a, rows_a, semg_a).start()
            sc_b.wait()
            if with_deg:
                dg_b.wait()
            idx_dst(cn_b, didx_b, semid_b).start()
            return carry
        lax.fori_loop(0, _STEPS // 2, pair, 0)

        pltpu.sync_copy(src.at[pl.ds(e0 + _STEPS * _K, _KT)], sidx_t)
        pltpu.sync_copy(dst.at[pl.ds(e0 + _STEPS * _K, _KT)], didx_t)
        pltpu.async_copy(h.at[sidx_t], rows_t, semg_b).wait()
        pltpu.sync_copy(rows_t, acc_sh.at[didx_t], add=True)
        if with_deg:
            pltpu.sync_copy(ones.at[pl.ds(0, _KT)], deg_sh.at[didx_t],
                            add=True)
        gath(sidx_a, rows_a, semg_a).wait()
        idx_dst(last, didx_a, semid_a).wait()
        idx_src(last, sidx_b, semis_b).wait()
        idx_dst(last, didx_b, semid_b).wait()
        plsc.subcore_barrier()

        ocps = [pltpu.async_copy(acc_sh.at[pl.ds(r0 + blk * _ZR, _ZR)],
                                 acc_out.at[c, pl.ds(r0 + blk * _ZR, _ZR), :],
                                 semg_b)
                for blk in range(_RPT // _ZR)]
        if with_deg:
            ocps.append(pltpu.async_copy(deg_sh.at[pl.ds(r0, _RPT)],
                                         deg_out.at[pl.ds(c * _NP + r0, _RPT)],
                                         semg_b))
        for cp in ocps:
            cp.wait()

    params = (pltpu.CompilerParams(use_tc_tiling_on_sc=False)
              if d % 128 else None)
    return pl.kernel(body, mesh=mesh, out_type=out_type,
                     scratch_types=scratch, compiler_params=params)


def _make_tc_layer(d_in: int, d_out: int, first: bool, ln: bool, act: bool,
                   proj: int = 0, pre_projected: bool = False,
                   bm: int = 400, n_rows: int = _N):
    grid = (n_rows // bm,)
    d_agg = d_out if pre_projected else d_in

    def body(*refs):
        refs = list(refs)
        h_ref = refs.pop(0)
        a_ref = refs.pop(0)
        nd_ref = refs.pop(0)
        wl_ref = refs.pop(0)
        wr_ref = None if pre_projected else refs.pop(0)
        b_ref = refs.pop(0)
        wp_ref = refs.pop(0) if proj else None
        o_ref = refs.pop(0)
        n_ref = refs.pop(0) if first else None
        p_ref = refs.pop(0) if proj else None
        if first:
            deg = nd_ref[0] + nd_ref[1]
            nrm = jnp.where(deg > 0, 1.0 / deg, 0.0)
            n_ref[...] = nrm
        else:
            nrm = nd_ref[...]
        ah = (a_ref[0] + a_ref[1]) * nrm
        out = jnp.dot(h_ref[...], wl_ref[...],
                      preferred_element_type=jnp.float32) + b_ref[...]
        if pre_projected:
            out = out + ah
        else:
            out = out + jnp.dot(ah, wr_ref[...],
                                preferred_element_type=jnp.float32)
        if ln:
            mu = jnp.mean(out, axis=1, keepdims=True)
            ctr = out - mu
            var = jnp.mean(ctr * ctr, axis=1, keepdims=True)
            out = ctr * lax.rsqrt(var + 1e-5)
        if act:
            out = jnp.maximum(out, 0.0)
        o_ref[...] = out
        if proj:
            p_ref[...] = jnp.dot(out, wp_ref[...],
                                 preferred_element_type=jnp.float32)

    in_specs = [
        pl.BlockSpec((bm, d_in), lambda i: (i, 0)),
        pl.BlockSpec((2, bm, d_agg), lambda i: (0, i, 0)),
        (pl.BlockSpec((2, bm, 1), lambda i: (0, i, 0)) if first
         else pl.BlockSpec((bm, 1), lambda i: (i, 0))),
        pl.BlockSpec((d_in, d_out), lambda i: (0, 0)),
    ]
    if not pre_projected:
        in_specs.append(pl.BlockSpec((d_in, d_out), lambda i: (0, 0)))
    in_specs.append(pl.BlockSpec((1, d_out), lambda i: (0, 0)))
    if proj:
        in_specs.append(pl.BlockSpec((d_out, proj), lambda i: (0, 0)))
    out_specs = [pl.BlockSpec((bm, d_out), lambda i: (i, 0))]
    out_shape = [jax.ShapeDtypeStruct((n_rows, d_out), jnp.float32)]
    if first:
        out_specs.append(pl.BlockSpec((bm, 1), lambda i: (i, 0)))
        out_shape.append(jax.ShapeDtypeStruct((n_rows, 1), jnp.float32))
    if proj:
        out_specs.append(pl.BlockSpec((bm, proj), lambda i: (i, 0)))
        out_shape.append(jax.ShapeDtypeStruct((n_rows, proj), jnp.float32))
    return pl.pallas_call(body, grid=grid, in_specs=in_specs,
                          out_specs=out_specs, out_shape=out_shape)


import functools as _functools

_make_sc_agg = _functools.cache(_make_sc_agg)
_LAYER1 = _make_tc_layer(_D, _D, first=True, ln=True, act=True)
_LAYER2 = _make_tc_layer(_D, _D, first=False, ln=True, act=True, proj=64)
_LAYER3 = _make_tc_layer(_D, 64, first=False, ln=False, act=False,
                         pre_projected=True)


def kernel(x, edge_index, W1, b1, W2, b2, W3, b3):
    src = edge_index[0]
    dst = edge_index[1]
    acc1, degf = _make_sc_agg(True)(src, dst, x)
    deg3 = degf.reshape(_NC, _NP, 1)
    h1, norm = _LAYER1(x, acc1, deg3, W1[:, :_D].T, W1[:, _D:].T,
                       b1.reshape(1, -1))
    acc2 = _make_sc_agg(False)(src, dst, h1)
    h2, p2 = _LAYER2(h1, acc2, norm, W2[:, :_D].T, W2[:, _D:].T,
                     b2.reshape(1, -1), W3[:, _D:].T)
    acc3 = _make_sc_agg(False, 64)(src, dst, p2)
    (h3,) = _LAYER3(h2, acc3, norm, W3[:, :_D].T, b3.reshape(1, -1))
    return h3
